# baseline (speedup 1.0000x reference)
import functools

import jax
import jax.numpy as jnp
from jax import lax
from jax.experimental import pallas as pl
from jax.experimental.pallas import tpu as pltpu
from jax.experimental.pallas import tpu_sc as plsc

B = 16384
E = 32
NT = 6
NC = 2
NS = 16
NW = NC * NS
B_PER_W = B // NW
GCHUNK = 128
NCHUNK = B_PER_W // GCHUNK


def _sc_gather(tables, idxs):
  mesh = plsc.VectorSubcoreMesh(core_axis_name="c", subcore_axis_name="s")
  out_types = [jax.ShapeDtypeStruct((B, E), jnp.float32)] * NT
  scratch = (
      [pltpu.VMEM((B_PER_W,), jnp.int32) for _ in range(NT)]
      + [pltpu.VMEM((B_PER_W, E), jnp.float32) for _ in range(NT)]
      + [pltpu.SemaphoreType.DMA]
      + [pltpu.SemaphoreType.DMA] * NT
      + [pltpu.SemaphoreType.DMA]
  )

  @functools.partial(pl.kernel, out_type=out_types, mesh=mesh,
                     scratch_types=scratch,
                     compiler_params=pltpu.CompilerParams(
                         use_tc_tiling_on_sc=False))
  def sc_kernel(*refs):
    t_hbm = refs[0:NT]
    i_hbm = refs[NT:2 * NT]
    o_hbm = refs[2 * NT:3 * NT]
    idx_v = refs[3 * NT:4 * NT]
    row_v = refs[4 * NT:5 * NT]
    sem_i = refs[5 * NT]
    sem_g = refs[5 * NT + 1:5 * NT + 1 + NT]
    sem_o = refs[5 * NT + 1 + NT]

    wid = lax.axis_index("s") * NC + lax.axis_index("c")
    base = wid * B_PER_W

    ih = [pltpu.async_copy(i_hbm[t].at[pl.ds(base, B_PER_W)], idx_v[t], sem_i)
          for t in range(NT)]
    gh = []
    for t in range(NT):
      ih[t].wait()
      gh.append([
          pltpu.async_copy(
              t_hbm[t].at[idx_v[t].at[pl.ds(j * GCHUNK, GCHUNK)]],
              row_v[t].at[pl.ds(j * GCHUNK, GCHUNK)],
              sem_g[t])
          for j in range(NCHUNK)
      ])
    oh = []
    for t in range(NT):
      for h in gh[t]:
        h.wait()
      oh.append(pltpu.async_copy(row_v[t], o_hbm[t].at[pl.ds(base, B_PER_W)],
                                 sem_o))
    for h in oh:
      h.wait()

  return sc_kernel(*tables, *idxs)


def _tc_mlp(es, W_lin, b_lin, W1, b1, W2, b2, W3, b3, W_out, b_out):
  blk = 2048

  def body(eu, em, eg, eo, ea, en, wlin, blin, w1, b1r, w2, b2r, w3, b3r,
           wout, bout, o_ref):
    e = (eu[...], em[...], eg[...], eo[...], ea[...], en[...])
    w1m = w1[...]
    wlinm = wlin[...]
    h = b1r[...][None, :] + jnp.zeros((blk, 128), jnp.float32)
    fm = blin[...][None, :] + jnp.zeros((blk, 1), jnp.float32)
    for t in range(NT):
      h = h + jnp.dot(e[t], w1m[t * E:(t + 1) * E, :],
                      preferred_element_type=jnp.float32)
      fm = fm + jnp.dot(e[t], wlinm[t * E:(t + 1) * E, :],
                        preferred_element_type=jnp.float32)
    h = jnp.maximum(h, 0.0)
    h = jnp.maximum(jnp.dot(h, w2[...], preferred_element_type=jnp.float32)
                    + b2r[...][None, :], 0.0)
    h = jnp.maximum(jnp.dot(h, w3[...], preferred_element_type=jnp.float32)
                    + b3r[...][None, :], 0.0)
    woutm = wout[...]
    out = (jnp.dot(h, woutm[0:32, :], preferred_element_type=jnp.float32)
           + fm * woutm[32:33, :] + bout[...][None, :])
    o_ref[...] = out

  e_spec = pl.BlockSpec((blk, E), lambda i: (i, 0))

  def full(x):
    shp = x.shape
    return pl.BlockSpec(shp, lambda i: tuple(0 for _ in shp))

  args = (W_lin, b_lin, W1, b1, W2, b2, W3, b3, W_out, b_out)
  out = pl.pallas_call(
      body,
      grid=(B // blk,),
      in_specs=[e_spec] * NT + [full(a) for a in args],
      out_specs=pl.BlockSpec((blk, 1), lambda i: (i, 0)),
      out_shape=jax.ShapeDtypeStruct((B, 1), jnp.float32),
  )(*es, *args)
  return out


def kernel(user, movie, gender, occupation, age, genres,
           user_table, movie_table, gender_table, occupation_table, age_table,
           genre_table, W_lin, b_lin, W1, b1, W2, b2, W3, b3, W_out, b_out):
  idxs = [x.astype(jnp.int32) for x in
          (user, movie, gender, occupation, age, genres)]
  tables = (user_table, movie_table, gender_table, occupation_table,
            age_table, genre_table)
  es = _sc_gather(tables, idxs)
  out = _tc_mlp(es, W_lin, b_lin, W1, b1, W2, b2, W3, b3, W_out, b_out)
  return out[:, 0]


# speedup vs baseline: 1.3267x; 1.3267x over previous
import functools

import jax
import jax.numpy as jnp
from jax import lax
from jax.experimental import pallas as pl
from jax.experimental.pallas import tpu as pltpu
from jax.experimental.pallas import tpu_sc as plsc

B = 16384
E = 32
NC = 2
NS = 16
NW = NC * NS
B_PER_W = B // NW
GCHUNK = 128
NCHUNK = B_PER_W // GCHUNK
NTAB = 3
SLOTS = 6


def _sc_gather3(tables, idxs):
  mesh = plsc.VectorSubcoreMesh(core_axis_name="c", subcore_axis_name="s")
  out_types = [jax.ShapeDtypeStruct((B, 128), jnp.float32)] * NTAB
  scratch = (
      [pltpu.VMEM((B_PER_W,), jnp.int32) for _ in range(NTAB)]
      + [pltpu.VMEM((GCHUNK, 128), jnp.float32) for _ in range(SLOTS)]
      + [pltpu.SemaphoreType.DMA]
      + [pltpu.SemaphoreType.DMA] * SLOTS
      + [pltpu.SemaphoreType.DMA] * SLOTS
  )

  @functools.partial(pl.kernel, out_type=out_types, mesh=mesh,
                     scratch_types=scratch,
                     compiler_params=pltpu.CompilerParams(
                         use_tc_tiling_on_sc=True))
  def sc_kernel(*refs):
    t_hbm = refs[0:NTAB]
    i_hbm = refs[NTAB:2 * NTAB]
    o_hbm = refs[2 * NTAB:3 * NTAB]
    idx_v = refs[3 * NTAB:4 * NTAB]
    bufs = refs[4 * NTAB:4 * NTAB + SLOTS]
    sem_i = refs[4 * NTAB + SLOTS]
    sem_g = refs[4 * NTAB + SLOTS + 1:4 * NTAB + 2 * SLOTS + 1]
    sem_o = refs[4 * NTAB + 2 * SLOTS + 1:4 * NTAB + 3 * SLOTS + 1]

    wid = lax.axis_index("s") * NC + lax.axis_index("c")
    base = wid * B_PER_W

    ih = [pltpu.async_copy(i_hbm[t].at[pl.ds(base, B_PER_W)], idx_v[t], sem_i)
          for t in range(NTAB)]
    for h in ih:
      h.wait()

    items = [(t, j) for t in range(NTAB) for j in range(NCHUNK)]
    n = len(items)
    gd = [None] * SLOTS
    od = [None] * SLOTS

    def fire_gather(k):
      t, j = items[k]
      s = k % SLOTS
      gd[s] = pltpu.async_copy(
          t_hbm[t].at[idx_v[t].at[pl.ds(j * GCHUNK, GCHUNK)]],
          bufs[s], sem_g[s])

    def fire_out(k):
      t, j = items[k]
      s = k % SLOTS
      gd[s].wait()
      od[s] = pltpu.async_copy(
          bufs[s], o_hbm[t].at[pl.ds(base + j * GCHUNK, GCHUNK)], sem_o[s])

    for k in range(n):
      s = k % SLOTS
      if k >= SLOTS:
        od[s].wait()
      fire_gather(k)
      if k >= SLOTS - 1:
        fire_out(k - (SLOTS - 1))
    for k in range(n - (SLOTS - 1), n):
      if k >= 0:
        fire_out(k)
    for h in od:
      if h is not None:
        h.wait()

  return sc_kernel(*tables, *idxs)


def _tc_mlp(wu, wm, ws, pu, pm, W_lin, b_lin, W1, b1, W2, b2, W3, b3,
            W_out, b_out):
  blk = 2048

  def body(wu_r, wm_r, ws_r, pu_r, pm_r, wlin, blin, w1, b1r, w2, b2r,
           w3, b3r, wout, bout, o_ref):
    wum = wu_r[...]
    wmm = wm_r[...]
    pum = pu_r[...][:, None]
    pmm = pm_r[...][:, None]
    eu = jnp.zeros((blk, E), jnp.float32)
    em = jnp.zeros((blk, E), jnp.float32)
    for c in range(4):
      eu = eu + jnp.where(pum == c, wum[:, c * E:(c + 1) * E], 0.0)
      em = em + jnp.where(pmm == c, wmm[:, c * E:(c + 1) * E], 0.0)
    es = ws_r[...]
    w1m = w1[...]
    wlinm = wlin[...]
    h = (jnp.dot(eu, w1m[0:E, :], preferred_element_type=jnp.float32)
         + jnp.dot(em, w1m[E:2 * E, :], preferred_element_type=jnp.float32)
         + jnp.dot(es, w1m[2 * E:, :], preferred_element_type=jnp.float32)
         + b1r[...][None, :])
    fm = (jnp.dot(eu, wlinm[0:E, :], preferred_element_type=jnp.float32)
          + jnp.dot(em, wlinm[E:2 * E, :], preferred_element_type=jnp.float32)
          + jnp.dot(es, wlinm[2 * E:, :], preferred_element_type=jnp.float32)
          + blin[...][None, :])
    h = jnp.maximum(h, 0.0)
    h = jnp.maximum(jnp.dot(h, w2[...], preferred_element_type=jnp.float32)
                    + b2r[...][None, :], 0.0)
    h = jnp.maximum(jnp.dot(h, w3[...], preferred_element_type=jnp.float32)
                    + b3r[...][None, :], 0.0)
    woutm = wout[...]
    out = (jnp.dot(h, woutm[0:32, :], preferred_element_type=jnp.float32)
           + fm * woutm[32:33, :] + bout[...][None, :])
    o_ref[...] = out

  wide_spec = pl.BlockSpec((blk, 128), lambda i: (i, 0))
  ph_spec = pl.BlockSpec((blk,), lambda i: (i,))

  def full(x):
    shp = x.shape
    return pl.BlockSpec(shp, lambda i: tuple(0 for _ in shp))

  args = (W_lin, b_lin, W1, b1, W2, b2, W3, b3, W_out, b_out)
  out = pl.pallas_call(
      body,
      grid=(B // blk,),
      in_specs=[wide_spec] * 3 + [ph_spec] * 2 + [full(a) for a in args],
      out_specs=pl.BlockSpec((blk, 1), lambda i: (i, 0)),
      out_shape=jax.ShapeDtypeStruct((B, 1), jnp.float32),
  )(wu, wm, ws, pu, pm, *args)
  return out


def kernel(user, movie, gender, occupation, age, genres,
           user_table, movie_table, gender_table, occupation_table, age_table,
           genre_table, W_lin, b_lin, W1, b1, W2, b2, W3, b3, W_out, b_out):
  iu = user.astype(jnp.int32)
  im = movie.astype(jnp.int32)
  ig = gender.astype(jnp.int32)
  io = occupation.astype(jnp.int32)
  ia = age.astype(jnp.int32)
  ie = genres.astype(jnp.int32)

  ut_packed = user_table.reshape(user_table.shape[0] // 4, 128)
  mt_packed = movie_table.reshape(movie_table.shape[0] // 4, 128)

  ng, no, na, ne = (gender_table.shape[0], occupation_table.shape[0],
                    age_table.shape[0], genre_table.shape[0])
  small_tab = jnp.concatenate([
      jnp.broadcast_to(gender_table[:, None, None, None, :],
                       (ng, no, na, ne, E)),
      jnp.broadcast_to(occupation_table[None, :, None, None, :],
                       (ng, no, na, ne, E)),
      jnp.broadcast_to(age_table[None, None, :, None, :],
                       (ng, no, na, ne, E)),
      jnp.broadcast_to(genre_table[None, None, None, :, :],
                       (ng, no, na, ne, E)),
  ], axis=-1).reshape(ng * no * na * ne, 128)
  small_idx = ((ig * no + io) * na + ia) * ne + ie

  wu, wm, ws = _sc_gather3(
      (ut_packed, mt_packed, small_tab),
      (iu // 4, im // 4, small_idx))
  out = _tc_mlp(wu, wm, ws, iu % 4, im % 4,
                W_lin, b_lin, W1, b1, W2, b2, W3, b3, W_out, b_out)
  return out[:, 0]


# speedup vs baseline: 2.3562x; 1.7760x over previous
import functools

import jax
import jax.numpy as jnp
from jax import lax
from jax.experimental import pallas as pl
from jax.experimental.pallas import tpu as pltpu
from jax.experimental.pallas import tpu_sc as plsc

B = 16384
E = 32
NC = 2
NS = 16
NW = NC * NS
B_PER_W = B // NW
GCHUNK = 128
NCHUNK = B_PER_W // GCHUNK
NTAB = 3
SLOTS = 6


def _sc_gather3(tables, idxs):
  mesh = plsc.VectorSubcoreMesh(core_axis_name="c", subcore_axis_name="s")
  out_types = [jax.ShapeDtypeStruct((B, 128), jnp.float32)] * NTAB
  scratch = (
      [pltpu.VMEM((B_PER_W,), jnp.int32) for _ in range(NTAB)]
      + [pltpu.VMEM((GCHUNK, 128), jnp.float32) for _ in range(SLOTS)]
      + [pltpu.SemaphoreType.DMA]
      + [pltpu.SemaphoreType.DMA] * SLOTS
      + [pltpu.SemaphoreType.DMA] * SLOTS
  )

  @functools.partial(pl.kernel, out_type=out_types, mesh=mesh,
                     scratch_types=scratch,
                     compiler_params=pltpu.CompilerParams(
                         use_tc_tiling_on_sc=True))
  def sc_kernel(*refs):
    t_hbm = refs[0:NTAB]
    i_hbm = refs[NTAB:2 * NTAB]
    o_hbm = refs[2 * NTAB:3 * NTAB]
    idx_v = refs[3 * NTAB:4 * NTAB]
    bufs = refs[4 * NTAB:4 * NTAB + SLOTS]
    sem_i = refs[4 * NTAB + SLOTS]
    sem_g = refs[4 * NTAB + SLOTS + 1:4 * NTAB + 2 * SLOTS + 1]
    sem_o = refs[4 * NTAB + 2 * SLOTS + 1:4 * NTAB + 3 * SLOTS + 1]

    wid = lax.axis_index("s") * NC + lax.axis_index("c")
    base = wid * B_PER_W

    ih = [pltpu.async_copy(i_hbm[t].at[pl.ds(base, B_PER_W)], idx_v[t], sem_i)
          for t in range(NTAB)]
    for h in ih:
      h.wait()

    items = [(t, j) for t in range(NTAB) for j in range(NCHUNK)]
    n = len(items)
    gd = [None] * SLOTS
    od = [None] * SLOTS

    def fire_gather(k):
      t, j = items[k]
      s = k % SLOTS
      gd[s] = pltpu.async_copy(
          t_hbm[t].at[idx_v[t].at[pl.ds(j * GCHUNK, GCHUNK)]],
          bufs[s], sem_g[s])

    def fire_out(k):
      t, j = items[k]
      s = k % SLOTS
      gd[s].wait()
      od[s] = pltpu.async_copy(
          bufs[s], o_hbm[t].at[pl.ds(base + j * GCHUNK, GCHUNK)], sem_o[s])

    for k in range(n):
      s = k % SLOTS
      if k >= SLOTS:
        od[s].wait()
      fire_gather(k)
      if k >= SLOTS - 1:
        fire_out(k - (SLOTS - 1))
    for k in range(n - (SLOTS - 1), n):
      if k >= 0:
        fire_out(k)
    for h in od:
      if h is not None:
        h.wait()

  return sc_kernel(*tables, *idxs)


def _tc_transpose_pad(tT):
  n = tT.shape[1]
  C = 8192
  grid = pl.cdiv(n, C)

  def body(x_ref, o_ref):
    o_ref[...] = jnp.pad(jnp.transpose(x_ref[...]), ((0, 0), (0, 128 - E)))

  return pl.pallas_call(
      body,
      grid=(grid,),
      in_specs=[pl.BlockSpec((E, C), lambda i: (0, i))],
      out_specs=pl.BlockSpec((C, 128), lambda i: (i, 0)),
      out_shape=jax.ShapeDtypeStruct((grid * C, 128), jnp.float32),
  )(tT)


def _tc_mlp(wu, wm, ws, W1u, W1m, W1s, Wlu, Wlm, Wls, b_lin, b1,
            W2, b2, W3, b3, W_out, b_out):
  blk = 2048

  def body(wu_r, wm_r, ws_r, w1u, w1m, w1s, wlu, wlm, wls, blin, b1r,
           w2, b2r, w3, b3r, wout, bout, o_ref):
    xu = wu_r[...]
    xm = wm_r[...]
    xs = ws_r[...]
    h = (jnp.dot(xu, w1u[...], preferred_element_type=jnp.float32)
         + jnp.dot(xm, w1m[...], preferred_element_type=jnp.float32)
         + jnp.dot(xs, w1s[...], preferred_element_type=jnp.float32)
         + b1r[...][None, :])
    fm = (jnp.dot(xu, wlu[...], preferred_element_type=jnp.float32)
          + jnp.dot(xm, wlm[...], preferred_element_type=jnp.float32)
          + jnp.dot(xs, wls[...], preferred_element_type=jnp.float32)
          + blin[...][None, :])
    h = jnp.maximum(h, 0.0)
    h = jnp.maximum(jnp.dot(h, w2[...], preferred_element_type=jnp.float32)
                    + b2r[...][None, :], 0.0)
    h = jnp.maximum(jnp.dot(h, w3[...], preferred_element_type=jnp.float32)
                    + b3r[...][None, :], 0.0)
    woutm = wout[...]
    out = (jnp.dot(h, woutm[0:32, :], preferred_element_type=jnp.float32)
           + fm * woutm[32:33, :] + bout[...][None, :])
    o_ref[...] = out

  wide_spec = pl.BlockSpec((blk, 128), lambda i: (i, 0))

  def full(x):
    shp = x.shape
    return pl.BlockSpec(shp, lambda i: tuple(0 for _ in shp))

  args = (W1u, W1m, W1s, Wlu, Wlm, Wls, b_lin, b1, W2, b2, W3, b3,
          W_out, b_out)
  out = pl.pallas_call(
      body,
      grid=(B // blk,),
      in_specs=[wide_spec] * 3 + [full(a) for a in args],
      out_specs=pl.BlockSpec((blk, 1), lambda i: (i, 0)),
      out_shape=jax.ShapeDtypeStruct((B, 1), jnp.float32),
  )(wu, wm, ws, *args)
  return out


def kernel(user, movie, gender, occupation, age, genres,
           user_table, movie_table, gender_table, occupation_table, age_table,
           genre_table, W_lin, b_lin, W1, b1, W2, b2, W3, b3, W_out, b_out):
  iu = user.astype(jnp.int32)
  im = movie.astype(jnp.int32)
  ig = gender.astype(jnp.int32)
  io = occupation.astype(jnp.int32)
  ia = age.astype(jnp.int32)
  ie = genres.astype(jnp.int32)

  ut_wide = _tc_transpose_pad(jnp.swapaxes(user_table, 0, 1))
  mt_wide = _tc_transpose_pad(jnp.swapaxes(movie_table, 0, 1))

  ng, no, na, ne = (gender_table.shape[0], occupation_table.shape[0],
                    age_table.shape[0], genre_table.shape[0])
  small_tab = jnp.concatenate([
      jnp.broadcast_to(gender_table[:, None, None, None, :],
                       (ng, no, na, ne, E)),
      jnp.broadcast_to(occupation_table[None, :, None, None, :],
                       (ng, no, na, ne, E)),
      jnp.broadcast_to(age_table[None, None, :, None, :],
                       (ng, no, na, ne, E)),
      jnp.broadcast_to(genre_table[None, None, None, :, :],
                       (ng, no, na, ne, E)),
  ], axis=-1).reshape(ng * no * na * ne, 128)
  small_idx = ((ig * no + io) * na + ia) * ne + ie

  wu, wm, ws = _sc_gather3((ut_wide, mt_wide, small_tab),
                           (iu, im, small_idx))

  zpad = ((0, 128 - E), (0, 0))
  W1u = jnp.pad(W1[0:E, :], zpad)
  W1m = jnp.pad(W1[E:2 * E, :], zpad)
  W1s = W1[2 * E:, :]
  Wlu = jnp.pad(W_lin[0:E, :], zpad)
  Wlm = jnp.pad(W_lin[E:2 * E, :], zpad)
  Wls = W_lin[2 * E:, :]

  out = _tc_mlp(wu, wm, ws, W1u, W1m, W1s, Wlu, Wlm, Wls, b_lin, b1,
                W2, b2, W3, b3, W_out, b_out)
  return out[:, 0]


# speedup vs baseline: 3.3938x; 1.4403x over previous
import functools

import jax
import jax.numpy as jnp
from jax import lax
from jax.experimental import pallas as pl
from jax.experimental.pallas import tpu as pltpu
from jax.experimental.pallas import tpu_sc as plsc

B = 16384
E = 32
NC = 2
NS = 16
NW = NC * NS
B_PER_W = B // NW
GCHUNK = 128
NCHUNK = B_PER_W // GCHUNK
NTAB = 3
SLOTS = 6
TC = 2048


def _tc_transpose_quarters(tT):
  n = tT.shape[1]
  nq = pl.cdiv(n, 4 * TC)
  tq = nq * TC

  def body(x0, x1, x2, x3, o_ref):
    x_all = jnp.concatenate([x[...] for x in (x0, x1, x2, x3)], axis=0)
    o_ref[...] = jnp.transpose(x_all)

  max_blk = (n - 1) // TC

  def make_spec(c):
    return pl.BlockSpec(
        (E, TC), lambda i, c=c: (0, jnp.minimum(c * nq + i, max_blk)))

  out = pl.pallas_call(
      body,
      grid=(nq,),
      in_specs=[make_spec(c) for c in range(4)],
      out_specs=pl.BlockSpec((TC, 128), lambda i: (i, 0)),
      out_shape=jax.ShapeDtypeStruct((tq, 128), jnp.float32),
  )(tT, tT, tT, tT)
  return out, tq


def _sc_gather3(tables, idxs):
  mesh = plsc.VectorSubcoreMesh(core_axis_name="c", subcore_axis_name="s")
  out_types = [jax.ShapeDtypeStruct((B, 128), jnp.float32)] * NTAB
  scratch = (
      [pltpu.VMEM((B_PER_W,), jnp.int32) for _ in range(NTAB)]
      + [pltpu.VMEM((GCHUNK, 128), jnp.float32) for _ in range(SLOTS)]
      + [pltpu.SemaphoreType.DMA]
      + [pltpu.SemaphoreType.DMA] * SLOTS
      + [pltpu.SemaphoreType.DMA] * SLOTS
  )

  @functools.partial(pl.kernel, out_type=out_types, mesh=mesh,
                     scratch_types=scratch,
                     compiler_params=pltpu.CompilerParams(
                         use_tc_tiling_on_sc=True))
  def sc_kernel(*refs):
    t_hbm = refs[0:NTAB]
    i_hbm = refs[NTAB:2 * NTAB]
    o_hbm = refs[2 * NTAB:3 * NTAB]
    idx_v = refs[3 * NTAB:4 * NTAB]
    bufs = refs[4 * NTAB:4 * NTAB + SLOTS]
    sem_i = refs[4 * NTAB + SLOTS]
    sem_g = refs[4 * NTAB + SLOTS + 1:4 * NTAB + 2 * SLOTS + 1]
    sem_o = refs[4 * NTAB + 2 * SLOTS + 1:4 * NTAB + 3 * SLOTS + 1]

    wid = lax.axis_index("s") * NC + lax.axis_index("c")
    base = wid * B_PER_W

    ih = [pltpu.async_copy(i_hbm[t].at[pl.ds(base, B_PER_W)], idx_v[t], sem_i)
          for t in range(NTAB)]
    for h in ih:
      h.wait()

    items = [(t, j) for t in range(NTAB) for j in range(NCHUNK)]
    n = len(items)
    gd = [None] * SLOTS
    od = [None] * SLOTS

    def fire_gather(k):
      t, j = items[k]
      s = k % SLOTS
      gd[s] = pltpu.async_copy(
          t_hbm[t].at[idx_v[t].at[pl.ds(j * GCHUNK, GCHUNK)]],
          bufs[s], sem_g[s])

    def fire_out(k):
      t, j = items[k]
      s = k % SLOTS
      gd[s].wait()
      od[s] = pltpu.async_copy(
          bufs[s], o_hbm[t].at[pl.ds(base + j * GCHUNK, GCHUNK)], sem_o[s])

    for k in range(n):
      s = k % SLOTS
      if k >= SLOTS:
        od[s].wait()
      fire_gather(k)
      if k >= SLOTS - 1:
        fire_out(k - (SLOTS - 1))
    for k in range(n - (SLOTS - 1), n):
      if k >= 0:
        fire_out(k)
    for h in od:
      if h is not None:
        h.wait()

  return sc_kernel(*tables, *idxs)


def _tc_mlp(wu, wm, ws, pu, pm, W1u, W1m, W1s, Wlu, Wlm, Wls, b_lin, b1,
            W2, b2, W3, b3, W_out, b_out):
  blk = 2048

  def body(wu_r, wm_r, ws_r, pu_r, pm_r, w1u, w1m, w1s, wlu, wlm, wls,
           blin, b1r, w2, b2r, w3, b3r, wout, bout, o_ref):
    wum = wu_r[...]
    wmm = wm_r[...]
    pum = pu_r[...][:, None]
    pmm = pm_r[...][:, None]
    xu = jnp.zeros((blk, E), jnp.float32)
    xm = jnp.zeros((blk, E), jnp.float32)
    for c in range(4):
      xu = xu + jnp.where(pum == c, wum[:, c * E:(c + 1) * E], 0.0)
      xm = xm + jnp.where(pmm == c, wmm[:, c * E:(c + 1) * E], 0.0)
    xs = ws_r[...]
    h = (jnp.dot(xu, w1u[...], preferred_element_type=jnp.float32)
         + jnp.dot(xm, w1m[...], preferred_element_type=jnp.float32)
         + jnp.dot(xs, w1s[...], preferred_element_type=jnp.float32)
         + b1r[...][None, :])
    fm = (jnp.dot(xu, wlu[...], preferred_element_type=jnp.float32)
          + jnp.dot(xm, wlm[...], preferred_element_type=jnp.float32)
          + jnp.dot(xs, wls[...], preferred_element_type=jnp.float32)
          + blin[...][None, :])
    h = jnp.maximum(h, 0.0)
    h = jnp.maximum(jnp.dot(h, w2[...], preferred_element_type=jnp.float32)
                    + b2r[...][None, :], 0.0)
    h = jnp.maximum(jnp.dot(h, w3[...], preferred_element_type=jnp.float32)
                    + b3r[...][None, :], 0.0)
    woutm = wout[...]
    out = (jnp.dot(h, woutm[0:32, :], preferred_element_type=jnp.float32)
           + fm * woutm[32:33, :] + bout[...][None, :])
    o_ref[...] = out

  wide_spec = pl.BlockSpec((blk, 128), lambda i: (i, 0))
  ph_spec = pl.BlockSpec((blk,), lambda i: (i,))

  def full(x):
    shp = x.shape
    return pl.BlockSpec(shp, lambda i: tuple(0 for _ in shp))

  args = (W1u, W1m, W1s, Wlu, Wlm, Wls, b_lin, b1, W2, b2, W3, b3,
          W_out, b_out)
  out = pl.pallas_call(
      body,
      grid=(B // blk,),
      in_specs=[wide_spec] * 3 + [ph_spec] * 2 + [full(a) for a in args],
      out_specs=pl.BlockSpec((blk, 1), lambda i: (i, 0)),
      out_shape=jax.ShapeDtypeStruct((B, 1), jnp.float32),
  )(wu, wm, ws, pu, pm, *args)
  return out


def kernel(user, movie, gender, occupation, age, genres,
           user_table, movie_table, gender_table, occupation_table, age_table,
           genre_table, W_lin, b_lin, W1, b1, W2, b2, W3, b3, W_out, b_out):
  iu = user.astype(jnp.int32)
  im = movie.astype(jnp.int32)
  ig = gender.astype(jnp.int32)
  io = occupation.astype(jnp.int32)
  ia = age.astype(jnp.int32)
  ie = genres.astype(jnp.int32)

  ut_q, tq_u = _tc_transpose_quarters(jnp.swapaxes(user_table, 0, 1))
  mt_q, tq_m = _tc_transpose_quarters(jnp.swapaxes(movie_table, 0, 1))

  ng, no, na, ne = (gender_table.shape[0], occupation_table.shape[0],
                    age_table.shape[0], genre_table.shape[0])
  small_tab = jnp.concatenate([
      jnp.broadcast_to(gender_table[:, None, None, None, :],
                       (ng, no, na, ne, E)),
      jnp.broadcast_to(occupation_table[None, :, None, None, :],
                       (ng, no, na, ne, E)),
      jnp.broadcast_to(age_table[None, None, :, None, :],
                       (ng, no, na, ne, E)),
      jnp.broadcast_to(genre_table[None, None, None, :, :],
                       (ng, no, na, ne, E)),
  ], axis=-1).reshape(ng * no * na * ne, 128)
  small_idx = ((ig * no + io) * na + ia) * ne + ie

  wu, wm, ws = _sc_gather3((ut_q, mt_q, small_tab),
                           (iu % tq_u, im % tq_m, small_idx))

  W1u = W1[0:E, :]
  W1m = W1[E:2 * E, :]
  W1s = W1[2 * E:, :]
  Wlu = W_lin[0:E, :]
  Wlm = W_lin[E:2 * E, :]
  Wls = W_lin[2 * E:, :]

  out = _tc_mlp(wu, wm, ws, iu // tq_u, im // tq_m,
                W1u, W1m, W1s, Wlu, Wlm, Wls,
                b_lin, b1, W2, b2, W3, b3, W_out, b_out)
  return out[:, 0]


# speedup vs baseline: 4.4219x; 1.3029x over previous
import functools

import jax
import jax.numpy as jnp
from jax import lax
from jax.experimental import pallas as pl
from jax.experimental.pallas import tpu as pltpu
from jax.experimental.pallas import tpu_sc as plsc

B = 16384
E = 32
NC = 2
NS = 16
NW = NC * NS
B_PER_W = B // NW
GCHUNK = 128
NCHUNK = B_PER_W // GCHUNK
NTAB = 3
SLOTS = 6
TC = 8192


def _tc_transpose_quarters(tT):
  n = tT.shape[1]
  nq = pl.cdiv(n, 4 * TC)
  tq = nq * TC

  def body(x0, x1, x2, x3, o_ref):
    x_all = jnp.concatenate([x[...] for x in (x0, x1, x2, x3)], axis=0)
    o_ref[...] = jnp.transpose(x_all)

  max_blk = (n - 1) // TC

  def make_spec(c):
    return pl.BlockSpec(
        (E, TC), lambda i, c=c: (0, jnp.minimum(c * nq + i, max_blk)))

  out = pl.pallas_call(
      body,
      grid=(nq,),
      in_specs=[make_spec(c) for c in range(4)],
      out_specs=pl.BlockSpec((TC, 128), lambda i: (i, 0)),
      out_shape=jax.ShapeDtypeStruct((tq, 128), jnp.float32),
  )(tT, tT, tT, tT)
  return out, tq


def _sc_gather3(tables, idxs):
  mesh = plsc.VectorSubcoreMesh(core_axis_name="c", subcore_axis_name="s")
  out_types = [jax.ShapeDtypeStruct((B, 128), jnp.float32)] * NTAB
  scratch = (
      [pltpu.VMEM((B_PER_W,), jnp.int32) for _ in range(NTAB)]
      + [pltpu.VMEM((GCHUNK, 128), jnp.float32) for _ in range(SLOTS)]
      + [pltpu.SemaphoreType.DMA]
      + [pltpu.SemaphoreType.DMA] * SLOTS
      + [pltpu.SemaphoreType.DMA] * SLOTS
  )

  @functools.partial(pl.kernel, out_type=out_types, mesh=mesh,
                     scratch_types=scratch,
                     compiler_params=pltpu.CompilerParams(
                         use_tc_tiling_on_sc=True))
  def sc_kernel(*refs):
    t_hbm = refs[0:NTAB]
    i_hbm = refs[NTAB:2 * NTAB]
    o_hbm = refs[2 * NTAB:3 * NTAB]
    idx_v = refs[3 * NTAB:4 * NTAB]
    bufs = refs[4 * NTAB:4 * NTAB + SLOTS]
    sem_i = refs[4 * NTAB + SLOTS]
    sem_g = refs[4 * NTAB + SLOTS + 1:4 * NTAB + 2 * SLOTS + 1]
    sem_o = refs[4 * NTAB + 2 * SLOTS + 1:4 * NTAB + 3 * SLOTS + 1]

    wid = lax.axis_index("s") * NC + lax.axis_index("c")
    base = wid * B_PER_W

    ih = [pltpu.async_copy(i_hbm[t].at[pl.ds(base, B_PER_W)], idx_v[t], sem_i)
          for t in range(NTAB)]
    for h in ih:
      h.wait()

    items = [(t, j) for t in range(NTAB) for j in range(NCHUNK)]
    n = len(items)
    gd = [None] * SLOTS
    od = [None] * SLOTS

    def fire_gather(k):
      t, j = items[k]
      s = k % SLOTS
      gd[s] = pltpu.async_copy(
          t_hbm[t].at[idx_v[t].at[pl.ds(j * GCHUNK, GCHUNK)]],
          bufs[s], sem_g[s])

    def fire_out(k):
      t, j = items[k]
      s = k % SLOTS
      gd[s].wait()
      od[s] = pltpu.async_copy(
          bufs[s], o_hbm[t].at[pl.ds(base + j * GCHUNK, GCHUNK)], sem_o[s])

    for k in range(n):
      s = k % SLOTS
      if k >= SLOTS:
        od[s].wait()
      fire_gather(k)
      if k >= SLOTS - 1:
        fire_out(k - (SLOTS - 1))
    for k in range(n - (SLOTS - 1), n):
      if k >= 0:
        fire_out(k)
    for h in od:
      if h is not None:
        h.wait()

  return sc_kernel(*tables, *idxs)


def _tc_mlp(wu, wm, ws, pu, pm, W_lin, b_lin, W1, b1,
            W2, b2, W3, b3, W_out, b_out):
  blk = 2048

  def body(wu_r, wm_r, ws_r, pu_r, pm_r, wlin, blin, w1, b1r,
           w2, b2r, w3, b3r, wout, bout, o_ref):
    wum = wu_r[...]
    wmm = wm_r[...]
    pum = pu_r[...][:, None]
    pmm = pm_r[...][:, None]
    xu = jnp.zeros((blk, E), jnp.float32)
    xm = jnp.zeros((blk, E), jnp.float32)
    for c in range(4):
      xu = xu + jnp.where(pum == c, wum[:, c * E:(c + 1) * E], 0.0)
      xm = xm + jnp.where(pmm == c, wmm[:, c * E:(c + 1) * E], 0.0)
    xs = ws_r[...]
    w1m_ = w1[...]
    wlinm = wlin[...]
    h = (jnp.dot(xu, w1m_[0:E, :], preferred_element_type=jnp.float32)
         + jnp.dot(xm, w1m_[E:2 * E, :], preferred_element_type=jnp.float32)
         + jnp.dot(xs, w1m_[2 * E:, :], preferred_element_type=jnp.float32)
         + b1r[...][None, :])
    fm = (jnp.dot(xu, wlinm[0:E, :], preferred_element_type=jnp.float32)
          + jnp.dot(xm, wlinm[E:2 * E, :], preferred_element_type=jnp.float32)
          + jnp.dot(xs, wlinm[2 * E:, :], preferred_element_type=jnp.float32)
          + blin[...][None, :])
    h = jnp.maximum(h, 0.0)
    h = jnp.maximum(jnp.dot(h, w2[...], preferred_element_type=jnp.float32)
                    + b2r[...][None, :], 0.0)
    h = jnp.maximum(jnp.dot(h, w3[...], preferred_element_type=jnp.float32)
                    + b3r[...][None, :], 0.0)
    woutm = wout[...]
    out = (jnp.dot(h, woutm[0:32, :], preferred_element_type=jnp.float32)
           + fm * woutm[32:33, :] + bout[...][None, :])
    o_ref[...] = out

  wide_spec = pl.BlockSpec((blk, 128), lambda i: (i, 0))
  ph_spec = pl.BlockSpec((blk,), lambda i: (i,))

  def full(x):
    shp = x.shape
    return pl.BlockSpec(shp, lambda i: tuple(0 for _ in shp))

  args = (W_lin, b_lin, W1, b1, W2, b2, W3, b3, W_out, b_out)
  out = pl.pallas_call(
      body,
      grid=(B // blk,),
      in_specs=[wide_spec] * 3 + [ph_spec] * 2 + [full(a) for a in args],
      out_specs=pl.BlockSpec((blk, 1), lambda i: (i, 0)),
      out_shape=jax.ShapeDtypeStruct((B, 1), jnp.float32),
  )(wu, wm, ws, pu, pm, *args)
  return out


def kernel(user, movie, gender, occupation, age, genres,
           user_table, movie_table, gender_table, occupation_table, age_table,
           genre_table, W_lin, b_lin, W1, b1, W2, b2, W3, b3, W_out, b_out):
  iu = user.astype(jnp.int32)
  im = movie.astype(jnp.int32)
  ig = gender.astype(jnp.int32)
  io = occupation.astype(jnp.int32)
  ia = age.astype(jnp.int32)
  ie = genres.astype(jnp.int32)

  ut_q, tq_u = _tc_transpose_quarters(jnp.swapaxes(user_table, 0, 1))
  mt_q, tq_m = _tc_transpose_quarters(jnp.swapaxes(movie_table, 0, 1))

  ng, no, na, ne = (gender_table.shape[0], occupation_table.shape[0],
                    age_table.shape[0], genre_table.shape[0])
  small_tab = jnp.concatenate([
      jnp.broadcast_to(gender_table[:, None, None, None, :],
                       (ng, no, na, ne, E)),
      jnp.broadcast_to(occupation_table[None, :, None, None, :],
                       (ng, no, na, ne, E)),
      jnp.broadcast_to(age_table[None, None, :, None, :],
                       (ng, no, na, ne, E)),
      jnp.broadcast_to(genre_table[None, None, None, :, :],
                       (ng, no, na, ne, E)),
  ], axis=-1).reshape(ng * no * na * ne, 128)
  small_idx = ((ig * no + io) * na + ia) * ne + ie

  wu, wm, ws = _sc_gather3((ut_q, mt_q, small_tab),
                           (iu % tq_u, im % tq_m, small_idx))

  out = _tc_mlp(wu, wm, ws, iu // tq_u, im // tq_m,
                W_lin, b_lin, W1, b1, W2, b2, W3, b3, W_out, b_out)
  return out[:, 0]


# speedup vs baseline: 4.4851x; 1.0143x over previous
import functools

import jax
import jax.numpy as jnp
from jax import lax
from jax.experimental import pallas as pl
from jax.experimental.pallas import tpu as pltpu
from jax.experimental.pallas import tpu_sc as plsc

B = 16384
E = 32
NC = 2
NS = 16
NW = NC * NS
B_PER_W = B // NW
GCHUNK = 128
NCHUNK = B_PER_W // GCHUNK
NTAB = 3
SLOTS = 6
TC = 16384


def _tc_transpose_quarters(tT):
  n = tT.shape[1]
  nq = pl.cdiv(n, 4 * TC)
  tq = nq * TC

  def body(x0, x1, x2, x3, o_ref):
    x_all = jnp.concatenate([x[...] for x in (x0, x1, x2, x3)], axis=0)
    o_ref[...] = jnp.transpose(x_all)

  max_blk = (n - 1) // TC

  def make_spec(c):
    return pl.BlockSpec(
        (E, TC), lambda i, c=c: (0, jnp.minimum(c * nq + i, max_blk)))

  out = pl.pallas_call(
      body,
      grid=(nq,),
      in_specs=[make_spec(c) for c in range(4)],
      out_specs=pl.BlockSpec((TC, 128), lambda i: (i, 0)),
      out_shape=jax.ShapeDtypeStruct((tq, 128), jnp.float32),
  )(tT, tT, tT, tT)
  return out, tq


def _sc_gather3(tables, idxs):
  mesh = plsc.VectorSubcoreMesh(core_axis_name="c", subcore_axis_name="s")
  out_types = [jax.ShapeDtypeStruct((B, 128), jnp.float32)] * NTAB
  scratch = (
      [pltpu.VMEM((B_PER_W,), jnp.int32) for _ in range(NTAB)]
      + [pltpu.VMEM((GCHUNK, 128), jnp.float32) for _ in range(SLOTS)]
      + [pltpu.SemaphoreType.DMA]
      + [pltpu.SemaphoreType.DMA] * SLOTS
      + [pltpu.SemaphoreType.DMA] * SLOTS
  )

  @functools.partial(pl.kernel, out_type=out_types, mesh=mesh,
                     scratch_types=scratch,
                     compiler_params=pltpu.CompilerParams(
                         use_tc_tiling_on_sc=True))
  def sc_kernel(*refs):
    t_hbm = refs[0:NTAB]
    i_hbm = refs[NTAB:2 * NTAB]
    o_hbm = refs[2 * NTAB:3 * NTAB]
    idx_v = refs[3 * NTAB:4 * NTAB]
    bufs = refs[4 * NTAB:4 * NTAB + SLOTS]
    sem_i = refs[4 * NTAB + SLOTS]
    sem_g = refs[4 * NTAB + SLOTS + 1:4 * NTAB + 2 * SLOTS + 1]
    sem_o = refs[4 * NTAB + 2 * SLOTS + 1:4 * NTAB + 3 * SLOTS + 1]

    wid = lax.axis_index("s") * NC + lax.axis_index("c")
    base = wid * B_PER_W

    ih = [pltpu.async_copy(i_hbm[t].at[pl.ds(base, B_PER_W)], idx_v[t], sem_i)
          for t in range(NTAB)]
    for h in ih:
      h.wait()

    items = [(t, j) for t in range(NTAB) for j in range(NCHUNK)]
    n = len(items)
    gd = [None] * SLOTS
    od = [None] * SLOTS

    def fire_gather(k):
      t, j = items[k]
      s = k % SLOTS
      gd[s] = pltpu.async_copy(
          t_hbm[t].at[idx_v[t].at[pl.ds(j * GCHUNK, GCHUNK)]],
          bufs[s], sem_g[s])

    def fire_out(k):
      t, j = items[k]
      s = k % SLOTS
      gd[s].wait()
      od[s] = pltpu.async_copy(
          bufs[s], o_hbm[t].at[pl.ds(base + j * GCHUNK, GCHUNK)], sem_o[s])

    for k in range(n):
      s = k % SLOTS
      if k >= SLOTS:
        od[s].wait()
      fire_gather(k)
      if k >= SLOTS - 1:
        fire_out(k - (SLOTS - 1))
    for k in range(n - (SLOTS - 1), n):
      if k >= 0:
        fire_out(k)
    for h in od:
      if h is not None:
        h.wait()

  return sc_kernel(*tables, *idxs)


def _tc_mlp(wu, wm, ws, pu, pm, W_lin, b_lin, W1, b1,
            W2, b2, W3, b3, W_out, b_out):
  blk = 4096

  def body(wu_r, wm_r, ws_r, pu_r, pm_r, wlin, blin, w1, b1r,
           w2, b2r, w3, b3r, wout, bout, o_ref):
    wum = wu_r[...]
    wmm = wm_r[...]
    pum = pu_r[...][:, None]
    pmm = pm_r[...][:, None]
    xu = jnp.zeros((blk, E), jnp.float32)
    xm = jnp.zeros((blk, E), jnp.float32)
    for c in range(4):
      xu = xu + jnp.where(pum == c, wum[:, c * E:(c + 1) * E], 0.0)
      xm = xm + jnp.where(pmm == c, wmm[:, c * E:(c + 1) * E], 0.0)
    xs = ws_r[...]
    w1m_ = w1[...]
    wlinm = wlin[...]
    h = (jnp.dot(xu, w1m_[0:E, :], preferred_element_type=jnp.float32)
         + jnp.dot(xm, w1m_[E:2 * E, :], preferred_element_type=jnp.float32)
         + jnp.dot(xs, w1m_[2 * E:, :], preferred_element_type=jnp.float32)
         + b1r[...][None, :])
    fm = (jnp.dot(xu, wlinm[0:E, :], preferred_element_type=jnp.float32)
          + jnp.dot(xm, wlinm[E:2 * E, :], preferred_element_type=jnp.float32)
          + jnp.dot(xs, wlinm[2 * E:, :], preferred_element_type=jnp.float32)
          + blin[...][None, :])
    h = jnp.maximum(h, 0.0)
    h = jnp.maximum(jnp.dot(h, w2[...], preferred_element_type=jnp.float32)
                    + b2r[...][None, :], 0.0)
    h = jnp.maximum(jnp.dot(h, w3[...], preferred_element_type=jnp.float32)
                    + b3r[...][None, :], 0.0)
    woutm = wout[...]
    out = (jnp.dot(h, woutm[0:32, :], preferred_element_type=jnp.float32)
           + fm * woutm[32:33, :] + bout[...][None, :])
    o_ref[...] = out

  wide_spec = pl.BlockSpec((blk, 128), lambda i: (i, 0))
  ph_spec = pl.BlockSpec((blk,), lambda i: (i,))

  def full(x):
    shp = x.shape
    return pl.BlockSpec(shp, lambda i: tuple(0 for _ in shp))

  args = (W_lin, b_lin, W1, b1, W2, b2, W3, b3, W_out, b_out)
  out = pl.pallas_call(
      body,
      grid=(B // blk,),
      in_specs=[wide_spec] * 3 + [ph_spec] * 2 + [full(a) for a in args],
      out_specs=pl.BlockSpec((blk, 1), lambda i: (i, 0)),
      out_shape=jax.ShapeDtypeStruct((B, 1), jnp.float32),
  )(wu, wm, ws, pu, pm, *args)
  return out


def kernel(user, movie, gender, occupation, age, genres,
           user_table, movie_table, gender_table, occupation_table, age_table,
           genre_table, W_lin, b_lin, W1, b1, W2, b2, W3, b3, W_out, b_out):
  iu = user.astype(jnp.int32)
  im = movie.astype(jnp.int32)
  ig = gender.astype(jnp.int32)
  io = occupation.astype(jnp.int32)
  ia = age.astype(jnp.int32)
  ie = genres.astype(jnp.int32)

  ut_q, tq_u = _tc_transpose_quarters(jnp.swapaxes(user_table, 0, 1))
  mt_q, tq_m = _tc_transpose_quarters(jnp.swapaxes(movie_table, 0, 1))

  ng, no, na, ne = (gender_table.shape[0], occupation_table.shape[0],
                    age_table.shape[0], genre_table.shape[0])
  small_tab = jnp.concatenate([
      jnp.broadcast_to(gender_table[:, None, None, None, :],
                       (ng, no, na, ne, E)),
      jnp.broadcast_to(occupation_table[None, :, None, None, :],
                       (ng, no, na, ne, E)),
      jnp.broadcast_to(age_table[None, None, :, None, :],
                       (ng, no, na, ne, E)),
      jnp.broadcast_to(genre_table[None, None, None, :, :],
                       (ng, no, na, ne, E)),
  ], axis=-1).reshape(ng * no * na * ne, 128)
  small_idx = ((ig * no + io) * na + ia) * ne + ie

  wu, wm, ws = _sc_gather3((ut_q, mt_q, small_tab),
                           (iu % tq_u, im % tq_m, small_idx))

  out = _tc_mlp(wu, wm, ws, iu // tq_u, im // tq_m,
                W_lin, b_lin, W1, b1, W2, b2, W3, b3, W_out, b_out)
  return out[:, 0]


# speedup vs baseline: 4.5449x; 1.0133x over previous
import functools

import jax
import jax.numpy as jnp
from jax import lax
from jax.experimental import pallas as pl
from jax.experimental.pallas import tpu as pltpu
from jax.experimental.pallas import tpu_sc as plsc

B = 16384
E = 32
NC = 2
NS = 16
NW = NC * NS
B_PER_W = B // NW
GCHUNK = 128
NCHUNK = B_PER_W // GCHUNK
NTAB = 3
SLOTS = 6
TC = 16384


def _tc_transpose_quarters(tT):
  n = tT.shape[1]
  nq = pl.cdiv(n, 4 * TC)
  tq = nq * TC

  def body(x0, x1, x2, x3, o_ref):
    x_all = jnp.concatenate([x[...] for x in (x0, x1, x2, x3)], axis=0)
    o_ref[...] = jnp.transpose(x_all)

  max_blk = (n - 1) // TC

  def make_spec(c):
    return pl.BlockSpec(
        (E, TC), lambda i, c=c: (0, jnp.minimum(c * nq + i, max_blk)))

  out = pl.pallas_call(
      body,
      grid=(nq,),
      in_specs=[make_spec(c) for c in range(4)],
      out_specs=pl.BlockSpec((TC, 128), lambda i: (i, 0)),
      out_shape=jax.ShapeDtypeStruct((tq, 128), jnp.float32),
  )(tT, tT, tT, tT)
  return out, tq


def _sc_gather(tables, idxs):
  ntab = len(tables)
  mesh = plsc.VectorSubcoreMesh(core_axis_name="c", subcore_axis_name="s")
  out_types = [jax.ShapeDtypeStruct((B, 128), jnp.float32)] * ntab
  scratch = (
      [pltpu.VMEM((B_PER_W,), jnp.int32) for _ in range(ntab)]
      + [pltpu.VMEM((GCHUNK, 128), jnp.float32) for _ in range(SLOTS)]
      + [pltpu.SemaphoreType.DMA]
      + [pltpu.SemaphoreType.DMA] * SLOTS
      + [pltpu.SemaphoreType.DMA] * SLOTS
  )

  @functools.partial(pl.kernel, out_type=out_types, mesh=mesh,
                     scratch_types=scratch,
                     compiler_params=pltpu.CompilerParams(
                         use_tc_tiling_on_sc=True))
  def sc_kernel(*refs):
    t_hbm = refs[0:ntab]
    i_hbm = refs[ntab:2 * ntab]
    o_hbm = refs[2 * ntab:3 * ntab]
    idx_v = refs[3 * ntab:4 * ntab]
    bufs = refs[4 * ntab:4 * ntab + SLOTS]
    sem_i = refs[4 * ntab + SLOTS]
    sem_g = refs[4 * ntab + SLOTS + 1:4 * ntab + 2 * SLOTS + 1]
    sem_o = refs[4 * ntab + 2 * SLOTS + 1:4 * ntab + 3 * SLOTS + 1]

    wid = lax.axis_index("s") * NC + lax.axis_index("c")
    base = wid * B_PER_W

    ih = [pltpu.async_copy(i_hbm[t].at[pl.ds(base, B_PER_W)], idx_v[t], sem_i)
          for t in range(ntab)]
    for h in ih:
      h.wait()

    items = [(t, j) for t in range(ntab) for j in range(NCHUNK)]
    n = len(items)
    gd = [None] * SLOTS
    od = [None] * SLOTS

    def fire_gather(k):
      t, j = items[k]
      s = k % SLOTS
      gd[s] = pltpu.async_copy(
          t_hbm[t].at[idx_v[t].at[pl.ds(j * GCHUNK, GCHUNK)]],
          bufs[s], sem_g[s])

    def fire_out(k):
      t, j = items[k]
      s = k % SLOTS
      gd[s].wait()
      od[s] = pltpu.async_copy(
          bufs[s], o_hbm[t].at[pl.ds(base + j * GCHUNK, GCHUNK)], sem_o[s])

    for k in range(n):
      s = k % SLOTS
      if k >= SLOTS:
        od[s].wait()
      fire_gather(k)
      if k >= SLOTS - 1:
        fire_out(k - (SLOTS - 1))
    for k in range(n - (SLOTS - 1), n):
      if k >= 0:
        fire_out(k)
    for h in od:
      if h is not None:
        h.wait()

  return sc_kernel(*tables, *idxs)


def _tc_mlp(wu, wm, ws, pu, pm, W_lin, b_lin, W1, b1,
            W2, b2, W3, b3, W_out, b_out):
  blk = 4096

  def body(wu_r, wm_r, ws_r, pu_r, pm_r, wlin, blin, w1, b1r,
           w2, b2r, w3, b3r, wout, bout, o_ref):
    wum = wu_r[...]
    wmm = wm_r[...]
    pum = pu_r[...][:, None]
    pmm = pm_r[...][:, None]
    xu = jnp.zeros((blk, E), jnp.float32)
    xm = jnp.zeros((blk, E), jnp.float32)
    for c in range(4):
      xu = xu + jnp.where(pum == c, wum[:, c * E:(c + 1) * E], 0.0)
      xm = xm + jnp.where(pmm == c, wmm[:, c * E:(c + 1) * E], 0.0)
    xs = ws_r[...]
    w1m_ = w1[...]
    wlinm = wlin[...]
    h = (jnp.dot(xu, w1m_[0:E, :], preferred_element_type=jnp.float32)
         + jnp.dot(xm, w1m_[E:2 * E, :], preferred_element_type=jnp.float32)
         + jnp.dot(xs, w1m_[2 * E:, :], preferred_element_type=jnp.float32)
         + b1r[...][None, :])
    fm = (jnp.dot(xu, wlinm[0:E, :], preferred_element_type=jnp.float32)
          + jnp.dot(xm, wlinm[E:2 * E, :], preferred_element_type=jnp.float32)
          + jnp.dot(xs, wlinm[2 * E:, :], preferred_element_type=jnp.float32)
          + blin[...][None, :])
    h = jnp.maximum(h, 0.0)
    h = jnp.maximum(jnp.dot(h, w2[...], preferred_element_type=jnp.float32)
                    + b2r[...][None, :], 0.0)
    h = jnp.maximum(jnp.dot(h, w3[...], preferred_element_type=jnp.float32)
                    + b3r[...][None, :], 0.0)
    woutm = wout[...]
    out = (jnp.dot(h, woutm[0:32, :], preferred_element_type=jnp.float32)
           + fm * woutm[32:33, :] + bout[...][None, :])
    o_ref[...] = out

  wide_spec = pl.BlockSpec((blk, 128), lambda i: (i, 0))
  ph_spec = pl.BlockSpec((blk,), lambda i: (i,))

  def full(x):
    shp = x.shape
    return pl.BlockSpec(shp, lambda i: tuple(0 for _ in shp))

  args = (W_lin, b_lin, W1, b1, W2, b2, W3, b3, W_out, b_out)
  out = pl.pallas_call(
      body,
      grid=(B // blk,),
      in_specs=[wide_spec] * 3 + [ph_spec] * 2 + [full(a) for a in args],
      out_specs=pl.BlockSpec((blk, 1), lambda i: (i, 0)),
      out_shape=jax.ShapeDtypeStruct((B, 1), jnp.float32),
  )(wu, wm, ws, pu, pm, *args)
  return out


def kernel(user, movie, gender, occupation, age, genres,
           user_table, movie_table, gender_table, occupation_table, age_table,
           genre_table, W_lin, b_lin, W1, b1, W2, b2, W3, b3, W_out, b_out):
  iu = user.astype(jnp.int32)
  im = movie.astype(jnp.int32)
  ig = gender.astype(jnp.int32)
  io = occupation.astype(jnp.int32)
  ia = age.astype(jnp.int32)
  ie = genres.astype(jnp.int32)

  mt_q, tq_m = _tc_transpose_quarters(jnp.swapaxes(movie_table, 0, 1))

  ng, no, na, ne = (gender_table.shape[0], occupation_table.shape[0],
                    age_table.shape[0], genre_table.shape[0])
  small_tab = jnp.concatenate([
      jnp.broadcast_to(gender_table[:, None, None, None, :],
                       (ng, no, na, ne, E)),
      jnp.broadcast_to(occupation_table[None, :, None, None, :],
                       (ng, no, na, ne, E)),
      jnp.broadcast_to(age_table[None, None, :, None, :],
                       (ng, no, na, ne, E)),
      jnp.broadcast_to(genre_table[None, None, None, :, :],
                       (ng, no, na, ne, E)),
  ], axis=-1).reshape(ng * no * na * ne, 128)
  small_idx = ((ig * no + io) * na + ia) * ne + ie

  wm, ws = _sc_gather((mt_q, small_tab), (im % tq_m, small_idx))
  ut_q, tq_u = _tc_transpose_quarters(jnp.swapaxes(user_table, 0, 1))
  (wu,) = _sc_gather((ut_q,), (iu % tq_u,))

  out = _tc_mlp(wu, wm, ws, iu // tq_u, im // tq_m,
                W_lin, b_lin, W1, b1, W2, b2, W3, b3, W_out, b_out)
  return out[:, 0]
